# Initial kernel scaffold; baseline (speedup 1.0000x reference)
import jax
import jax.numpy as jnp
from jax.experimental import pallas as pl


def kernel(x, edge_index, batch, W1_l, W1_r, b1, W2_l, W2_r, b2, g1, be1, g2, be2, Wc1, bc1, gc, bec, Wc2, bc2):
    raise NotImplementedError("write your pallas kernel here")


# speedup vs baseline: 5.8655x; 5.8655x over previous
import functools

import jax
import jax.numpy as jnp
from jax import lax
from jax.experimental import pallas as pl
from jax.experimental.pallas import tpu as pltpu
from jax.experimental.pallas import tpu_sc as plsc

N_NODES = 10000
N_EDGES = 320000
D_FEAT = 128
HIDDEN = 64
N_CLASSES = 10
NUM_GRAPHS = 64

_NC = 2
_NS = 16
_NW = _NC * _NS
_EPT = N_EDGES // _NW
_CH = 80
_NCH = _EPT // _CH
_NPAD = 10240
_SLAB = _NPAD // _NS


@functools.lru_cache(maxsize=None)
def _make_sc_agg(width):
    mesh = plsc.VectorSubcoreMesh(core_axis_name="c", subcore_axis_name="s")

    @functools.partial(
        pl.kernel,
        mesh=mesh,
        compiler_params=pltpu.CompilerParams(use_tc_tiling_on_sc=False),
        out_type=jax.ShapeDtypeStruct((_NC, _NPAD, width), jnp.float32),
        scratch_types=[
            pltpu.VMEM_SHARED((_NPAD, width), jnp.float32),
            pltpu.VMEM((_SLAB, width), jnp.float32),
            pltpu.VMEM((_CH, width), jnp.float32),
            pltpu.VMEM((_CH,), jnp.int32),
            pltpu.VMEM((_CH,), jnp.int32),
        ],
    )
    def agg(tbl_hbm, src_hbm, dst_hbm, zrow_hbm, out_hbm,
            acc_sh, vbuf, rows_v, src_v, dst_v):
        c = lax.axis_index("c")
        s = lax.axis_index("s")
        slab = s * _SLAB
        pltpu.sync_copy(zrow_hbm, vbuf)
        pltpu.sync_copy(vbuf, acc_sh.at[pl.ds(slab, _SLAB)])
        plsc.subcore_barrier()

        ebase = (c * _NS + s) * _EPT

        def step(i, carry):
            off = ebase + i * _CH
            pltpu.sync_copy(src_hbm.at[pl.ds(off, _CH)], src_v)
            pltpu.sync_copy(dst_hbm.at[pl.ds(off, _CH)], dst_v)
            pltpu.sync_copy(tbl_hbm.at[src_v], rows_v)
            pltpu.sync_copy(rows_v, acc_sh.at[dst_v], add=True)
            return carry

        lax.fori_loop(0, _NCH, step, 0)
        plsc.subcore_barrier()
        pltpu.sync_copy(acc_sh.at[pl.ds(slab, _SLAB)], vbuf)
        pltpu.sync_copy(vbuf, out_hbm.at[c, pl.ds(slab, _SLAB)])

    return agg


def _tc1_body(x_ref, wl_ref, wr_ref, b1_ref, y80_ref, yr_ref):
    x = x_ref[...]
    y = jnp.dot(x, wl_ref[...], preferred_element_type=jnp.float32,
                precision=lax.Precision.HIGHEST)
    ones = jnp.ones((N_NODES, 16), jnp.float32)
    y80 = jnp.concatenate([y, ones], axis=1)
    pad = jnp.zeros((_NPAD - N_NODES, HIDDEN + 16), jnp.float32)
    y80_ref[...] = jnp.concatenate([y80, pad], axis=0)
    yr_ref[...] = jnp.dot(x, wr_ref[...], preferred_element_type=jnp.float32,
                precision=lax.Precision.HIGHEST) + b1_ref[...]


def _bn_relu(h, g, b):
    mu = jnp.mean(h, axis=0, keepdims=True)
    var = jnp.mean((h - mu) * (h - mu), axis=0, keepdims=True)
    return jnp.maximum((h - mu) * lax.rsqrt(var + 1e-5) * g + b, 0.0)


def _tc2_body(sums_ref, yr_ref, g1_ref, be1_ref, wl_ref, wr_ref, b2_ref,
              z80_ref, zr_ref, r64_ref):
    s = sums_ref[0, :N_NODES, :] + sums_ref[1, :N_NODES, :]
    cnt = s[:, HIDDEN:HIDDEN + 16]
    r = 1.0 / jnp.maximum(cnt, 1.0)
    r64 = jnp.concatenate([r, r, r, r], axis=1)
    h = s[:, :HIDDEN] * r64 + yr_ref[...]
    h = _bn_relu(h, g1_ref[...], be1_ref[...])
    z = jnp.dot(h, wl_ref[...], preferred_element_type=jnp.float32,
                precision=lax.Precision.HIGHEST)
    pad = jnp.zeros((_NPAD - N_NODES, HIDDEN), jnp.float32)
    z80_ref[...] = jnp.concatenate([z, pad], axis=0)
    zr_ref[...] = jnp.dot(h, wr_ref[...], preferred_element_type=jnp.float32,
                precision=lax.Precision.HIGHEST) + b2_ref[...]
    r64_ref[...] = r64


def _tc3_body(sums_ref, zr_ref, r64_ref, g2_ref, be2_ref, batch_ref,
              wc1_ref, bc1_ref, gc_ref, bec_ref, wc2_ref, bc2_ref, out_ref):
    s = sums_ref[0, :N_NODES, :] + sums_ref[1, :N_NODES, :]
    h = s * r64_ref[...] + zr_ref[...]
    h = _bn_relu(h, g2_ref[...], be2_ref[...])
    gids = lax.broadcasted_iota(jnp.int32, (NUM_GRAPHS, N_NODES), 0)
    oh = (gids == batch_ref[...]).astype(jnp.float32)
    pooled = jnp.dot(oh, h, preferred_element_type=jnp.float32,
                precision=lax.Precision.HIGHEST)
    cg = jnp.sum(oh, axis=1, keepdims=True)
    pooled = pooled / jnp.maximum(cg, 1.0)
    hc = jnp.dot(pooled, wc1_ref[...], preferred_element_type=jnp.float32,
                precision=lax.Precision.HIGHEST) + bc1_ref[...]
    mu = jnp.mean(hc, axis=0, keepdims=True)
    var = jnp.mean((hc - mu) * (hc - mu), axis=0, keepdims=True)
    hc = jnp.maximum((hc - mu) * lax.rsqrt(var + 1e-5) * gc_ref[...] + bec_ref[...], 0.0)
    logits = jnp.dot(hc, wc2_ref[...], preferred_element_type=jnp.float32,
                precision=lax.Precision.HIGHEST) + bc2_ref[...]
    m = jnp.max(logits, axis=1, keepdims=True)
    lse = m + jnp.log(jnp.sum(jnp.exp(logits - m), axis=1, keepdims=True))
    out_ref[...] = logits - lse


def kernel(x, edge_index, batch, W1_l, W1_r, b1, W2_l, W2_r, b2,
           g1, be1, g2, be2, Wc1, bc1, gc, bec, Wc2, bc2):
    f32 = jnp.float32
    src = edge_index[0].astype(jnp.int32)
    dst = edge_index[1].astype(jnp.int32)
    batch2d = batch.astype(jnp.int32).reshape(1, N_NODES)
    z80 = jnp.zeros((_SLAB, HIDDEN + 16), f32)
    z64 = jnp.zeros((_SLAB, HIDDEN), f32)

    y80, y1r = pl.pallas_call(
        _tc1_body,
        out_shape=[jax.ShapeDtypeStruct((_NPAD, HIDDEN + 16), f32),
                   jax.ShapeDtypeStruct((N_NODES, HIDDEN), f32)],
    )(x, W1_l, W1_r, b1.reshape(1, HIDDEN))

    sums1 = _make_sc_agg(HIDDEN + 16)(y80, src, dst, z80)

    z2l, z2r, r64 = pl.pallas_call(
        _tc2_body,
        out_shape=[jax.ShapeDtypeStruct((_NPAD, HIDDEN), f32),
                   jax.ShapeDtypeStruct((N_NODES, HIDDEN), f32),
                   jax.ShapeDtypeStruct((N_NODES, HIDDEN), f32)],
    )(sums1, y1r, g1.reshape(1, HIDDEN), be1.reshape(1, HIDDEN),
      W2_l, W2_r, b2.reshape(1, HIDDEN))

    sums2 = _make_sc_agg(HIDDEN)(z2l, src, dst, z64)

    out = pl.pallas_call(
        _tc3_body,
        out_shape=jax.ShapeDtypeStruct((NUM_GRAPHS, N_CLASSES), f32),
    )(sums2, z2r, r64, g2.reshape(1, HIDDEN), be2.reshape(1, HIDDEN),
      batch2d, Wc1, bc1.reshape(1, HIDDEN), gc.reshape(1, HIDDEN),
      bec.reshape(1, HIDDEN), Wc2, bc2.reshape(1, N_CLASSES))
    return out


# speedup vs baseline: 9.6000x; 1.6367x over previous
import functools

import jax
import jax.numpy as jnp
from jax import lax
from jax.experimental import pallas as pl
from jax.experimental.pallas import tpu as pltpu
from jax.experimental.pallas import tpu_sc as plsc

N_NODES = 10000
N_EDGES = 320000
D_FEAT = 128
HIDDEN = 64
N_CLASSES = 10
NUM_GRAPHS = 64

_NC = 2
_NS = 16
_NW = _NC * _NS
_EPT = N_EDGES // _NW
_CH = 80
_NCH = _EPT // _CH
_NPAIRS = _NCH // 2
_NPAD = 10240
_SLAB = _NPAD // _NS


@functools.lru_cache(maxsize=None)
def _make_sc_agg(width):
    mesh = plsc.VectorSubcoreMesh(core_axis_name="c", subcore_axis_name="s")

    @functools.partial(
        pl.kernel,
        mesh=mesh,
        compiler_params=pltpu.CompilerParams(use_tc_tiling_on_sc=False),
        out_type=jax.ShapeDtypeStruct((_NC, _NPAD, width), jnp.float32),
        scratch_types=[
            pltpu.VMEM_SHARED((_NPAD, width), jnp.float32),
            pltpu.VMEM((_SLAB, width), jnp.float32),
            pltpu.VMEM((_CH,), jnp.int32),
            pltpu.VMEM((_CH,), jnp.int32),
            pltpu.VMEM((_CH,), jnp.int32),
            pltpu.VMEM((_CH,), jnp.int32),
            pltpu.VMEM((_CH, width), jnp.float32),
            pltpu.VMEM((_CH, width), jnp.float32),
            pltpu.SemaphoreType.DMA,
            pltpu.SemaphoreType.DMA,
            pltpu.SemaphoreType.DMA,
            pltpu.SemaphoreType.DMA,
            pltpu.SemaphoreType.DMA,
            pltpu.SemaphoreType.DMA,
        ],
    )
    def agg(tbl_hbm, src_hbm, dst_hbm, zrow_hbm, out_hbm,
            acc_sh, vbuf, si0, di0, si1, di1, r0, r1,
            smi0, smi1, sg0, sg1, ss0, ss1):
        c = lax.axis_index("c")
        s = lax.axis_index("s")
        slab = s * _SLAB
        ebase = (c * _NS + s) * _EPT
        pltpu.sync_copy(zrow_hbm, vbuf)
        pltpu.sync_copy(vbuf, acc_sh.at[pl.ds(slab, _SLAB)])
        plsc.subcore_barrier()

        def idx_load(i, sref, dref, sem):
            off = ebase + i * _CH
            pltpu.async_copy(src_hbm.at[pl.ds(off, _CH)], sref, sem)
            pltpu.async_copy(dst_hbm.at[pl.ds(off, _CH)], dref, sem)

        def idx_wait(i, sref, dref, sem):
            off = ebase + i * _CH
            pltpu.make_async_copy(src_hbm.at[pl.ds(off, _CH)], sref, sem).wait()
            pltpu.make_async_copy(dst_hbm.at[pl.ds(off, _CH)], dref, sem).wait()

        idx_load(0, si0, di0, smi0)
        idx_load(1, si1, di1, smi1)
        idx_wait(0, si0, di0, smi0)
        pltpu.async_copy(tbl_hbm.at[si0], r0, sg0)
        last = _NPAIRS - 1

        def step(j, carry):
            i = 2 * j
            pltpu.make_async_copy(tbl_hbm.at[si0], r0, sg0).wait()
            sc0 = pltpu.async_copy(r0, acc_sh.at[di0], ss0, add=True)
            idx_wait(i + 1, si1, di1, smi1)
            pltpu.async_copy(tbl_hbm.at[si1], r1, sg1)
            sc0.wait()

            @pl.when(j < last)
            def _():
                idx_load(i + 2, si0, di0, smi0)

            pltpu.make_async_copy(tbl_hbm.at[si1], r1, sg1).wait()
            sc1 = pltpu.async_copy(r1, acc_sh.at[di1], ss1, add=True)

            @pl.when(j < last)
            def _():
                idx_wait(i + 2, si0, di0, smi0)
                pltpu.async_copy(tbl_hbm.at[si0], r0, sg0)

            sc1.wait()

            @pl.when(j < last)
            def _():
                idx_load(i + 3, si1, di1, smi1)

            return carry

        lax.fori_loop(0, _NPAIRS, step, 0)
        i = _NCH - 1
        idx_load(i, si0, di0, smi0)
        idx_wait(i, si0, di0, smi0)
        pltpu.sync_copy(tbl_hbm.at[si0], r0)
        pltpu.sync_copy(r0, acc_sh.at[di0], add=True)
        plsc.subcore_barrier()
        pltpu.sync_copy(acc_sh.at[pl.ds(slab, _SLAB)], vbuf)
        pltpu.sync_copy(vbuf, out_hbm.at[c, pl.ds(slab, _SLAB)])

    return agg


def _tc1_body(x_ref, wl_ref, wr_ref, b1_ref, y80_ref, yr_ref):
    x = x_ref[...]
    y = jnp.dot(x, wl_ref[...], preferred_element_type=jnp.float32,
                precision=lax.Precision.HIGHEST)
    ones = jnp.ones((N_NODES, 16), jnp.float32)
    y80 = jnp.concatenate([y, ones], axis=1)
    pad = jnp.zeros((_NPAD - N_NODES, HIDDEN + 16), jnp.float32)
    y80_ref[...] = jnp.concatenate([y80, pad], axis=0)
    yr_ref[...] = jnp.dot(x, wr_ref[...], preferred_element_type=jnp.float32,
                precision=lax.Precision.HIGHEST) + b1_ref[...]


def _bn_relu(h, g, b):
    mu = jnp.mean(h, axis=0, keepdims=True)
    var = jnp.mean((h - mu) * (h - mu), axis=0, keepdims=True)
    return jnp.maximum((h - mu) * lax.rsqrt(var + 1e-5) * g + b, 0.0)


def _tc2_body(sums_ref, yr_ref, g1_ref, be1_ref, wl_ref, wr_ref, b2_ref,
              z80_ref, zr_ref):
    s = sums_ref[0, :N_NODES, :] + sums_ref[1, :N_NODES, :]
    cnt = s[:, HIDDEN:HIDDEN + 16]
    r = 1.0 / jnp.maximum(cnt, 1.0)
    r64 = jnp.concatenate([r, r, r, r], axis=1)
    h = s[:, :HIDDEN] * r64 + yr_ref[...]
    h = _bn_relu(h, g1_ref[...], be1_ref[...])
    z = jnp.dot(h, wl_ref[...], preferred_element_type=jnp.float32,
                precision=lax.Precision.HIGHEST)
    ones = jnp.ones((N_NODES, 16), jnp.float32)
    pad = jnp.zeros((_NPAD - N_NODES, HIDDEN + 16), jnp.float32)
    z80_ref[...] = jnp.concatenate([jnp.concatenate([z, ones], axis=1), pad], axis=0)
    zr_ref[...] = jnp.dot(h, wr_ref[...], preferred_element_type=jnp.float32,
                precision=lax.Precision.HIGHEST) + b2_ref[...]


def _tc3_body(sums_ref, zr_ref, g2_ref, be2_ref, batch_ref,
              wc1_ref, bc1_ref, gc_ref, bec_ref, wc2_ref, bc2_ref, out_ref):
    s = sums_ref[0, :N_NODES, :] + sums_ref[1, :N_NODES, :]
    cnt = s[:, HIDDEN:HIDDEN + 16]
    r = 1.0 / jnp.maximum(cnt, 1.0)
    r64 = jnp.concatenate([r, r, r, r], axis=1)
    h = s[:, :HIDDEN] * r64 + zr_ref[...]
    h = _bn_relu(h, g2_ref[...], be2_ref[...])
    gids = lax.broadcasted_iota(jnp.int32, (NUM_GRAPHS, N_NODES), 0)
    oh = (gids == batch_ref[...]).astype(jnp.float32)
    pooled = jnp.dot(oh, h, preferred_element_type=jnp.float32,
                precision=lax.Precision.HIGHEST)
    cg = jnp.sum(oh, axis=1, keepdims=True)
    pooled = pooled / jnp.maximum(cg, 1.0)
    hc = jnp.dot(pooled, wc1_ref[...], preferred_element_type=jnp.float32,
                precision=lax.Precision.HIGHEST) + bc1_ref[...]
    mu = jnp.mean(hc, axis=0, keepdims=True)
    var = jnp.mean((hc - mu) * (hc - mu), axis=0, keepdims=True)
    hc = jnp.maximum((hc - mu) * lax.rsqrt(var + 1e-5) * gc_ref[...] + bec_ref[...], 0.0)
    logits = jnp.dot(hc, wc2_ref[...], preferred_element_type=jnp.float32,
                precision=lax.Precision.HIGHEST) + bc2_ref[...]
    m = jnp.max(logits, axis=1, keepdims=True)
    lse = m + jnp.log(jnp.sum(jnp.exp(logits - m), axis=1, keepdims=True))
    out_ref[...] = logits - lse


def kernel(x, edge_index, batch, W1_l, W1_r, b1, W2_l, W2_r, b2,
           g1, be1, g2, be2, Wc1, bc1, gc, bec, Wc2, bc2):
    f32 = jnp.float32
    src = edge_index[0].astype(jnp.int32)
    dst = edge_index[1].astype(jnp.int32)
    batch2d = batch.astype(jnp.int32).reshape(1, N_NODES)
    zrow = jnp.zeros((_SLAB, HIDDEN + 16), f32)

    y80, y1r = pl.pallas_call(
        _tc1_body,
        out_shape=[jax.ShapeDtypeStruct((_NPAD, HIDDEN + 16), f32),
                   jax.ShapeDtypeStruct((N_NODES, HIDDEN), f32)],
    )(x, W1_l, W1_r, b1.reshape(1, HIDDEN))

    sums1 = _make_sc_agg(HIDDEN + 16)(y80, src, dst, zrow)

    z80, z2r = pl.pallas_call(
        _tc2_body,
        out_shape=[jax.ShapeDtypeStruct((_NPAD, HIDDEN + 16), f32),
                   jax.ShapeDtypeStruct((N_NODES, HIDDEN), f32)],
    )(sums1, y1r, g1.reshape(1, HIDDEN), be1.reshape(1, HIDDEN),
      W2_l, W2_r, b2.reshape(1, HIDDEN))

    sums2 = _make_sc_agg(HIDDEN + 16)(z80, src, dst, zrow)

    out = pl.pallas_call(
        _tc3_body,
        out_shape=jax.ShapeDtypeStruct((NUM_GRAPHS, N_CLASSES), f32),
    )(sums2, z2r, g2.reshape(1, HIDDEN), be2.reshape(1, HIDDEN),
      batch2d, Wc1, bc1.reshape(1, HIDDEN), gc.reshape(1, HIDDEN),
      bec.reshape(1, HIDDEN), Wc2, bc2.reshape(1, N_CLASSES))
    return out


# speedup vs baseline: 10.0848x; 1.0505x over previous
import functools

import jax
import jax.numpy as jnp
from jax import lax
from jax.experimental import pallas as pl
from jax.experimental.pallas import tpu as pltpu
from jax.experimental.pallas import tpu_sc as plsc

N_NODES = 10000
N_EDGES = 320000
D_FEAT = 128
HIDDEN = 64
N_CLASSES = 10
NUM_GRAPHS = 64

_NC = 2
_NS = 16
_NW = _NC * _NS
_EPT = N_EDGES // _NW
_CH = 80
_NCH = _EPT // _CH
_NPAIRS = _NCH // 2
_NPAD = 10240
_SLAB = _NPAD // _NS


@functools.lru_cache(maxsize=None)
def _make_sc_agg(width):
    mesh = plsc.VectorSubcoreMesh(core_axis_name="c", subcore_axis_name="s")

    @functools.partial(
        pl.kernel,
        mesh=mesh,
        compiler_params=pltpu.CompilerParams(use_tc_tiling_on_sc=False),
        out_type=jax.ShapeDtypeStruct((_NC, _NPAD, width), jnp.float32),
        scratch_types=[
            pltpu.VMEM_SHARED((_NPAD, width), jnp.float32),
            pltpu.VMEM((_SLAB, width), jnp.float32),
            pltpu.VMEM((_CH,), jnp.int32),
            pltpu.VMEM((_CH,), jnp.int32),
            pltpu.VMEM((_CH,), jnp.int32),
            pltpu.VMEM((_CH,), jnp.int32),
            pltpu.VMEM((_CH, width), jnp.float32),
            pltpu.VMEM((_CH, width), jnp.float32),
            pltpu.SemaphoreType.DMA,
            pltpu.SemaphoreType.DMA,
            pltpu.SemaphoreType.DMA,
            pltpu.SemaphoreType.DMA,
            pltpu.SemaphoreType.DMA,
            pltpu.SemaphoreType.DMA,
        ],
    )
    def agg(tbl_hbm, src_hbm, dst_hbm, zrow_hbm, out_hbm,
            acc_sh, vbuf, si0, di0, si1, di1, r0, r1,
            smi0, smi1, sg0, sg1, ss0, ss1):
        c = lax.axis_index("c")
        s = lax.axis_index("s")
        slab = s * _SLAB
        ebase = (c * _NS + s) * _EPT
        pltpu.sync_copy(zrow_hbm, vbuf)
        pltpu.sync_copy(vbuf, acc_sh.at[pl.ds(slab, _SLAB)])
        plsc.subcore_barrier()

        def idx_load(i, sref, dref, sem):
            off = ebase + i * _CH
            pltpu.async_copy(src_hbm.at[pl.ds(off, _CH)], sref, sem)
            pltpu.async_copy(dst_hbm.at[pl.ds(off, _CH)], dref, sem)

        def idx_wait(i, sref, dref, sem):
            off = ebase + i * _CH
            pltpu.make_async_copy(src_hbm.at[pl.ds(off, _CH)], sref, sem).wait()
            pltpu.make_async_copy(dst_hbm.at[pl.ds(off, _CH)], dref, sem).wait()

        idx_load(0, si0, di0, smi0)
        idx_load(1, si1, di1, smi1)
        idx_wait(0, si0, di0, smi0)
        pltpu.async_copy(tbl_hbm.at[si0], r0, sg0)
        last = _NPAIRS - 1

        def step(j, carry):
            i = 2 * j
            pltpu.make_async_copy(tbl_hbm.at[si0], r0, sg0).wait()
            sc0 = pltpu.async_copy(r0, acc_sh.at[di0], ss0, add=True)
            idx_wait(i + 1, si1, di1, smi1)
            pltpu.async_copy(tbl_hbm.at[si1], r1, sg1)
            sc0.wait()

            @pl.when(j < last)
            def _():
                idx_load(i + 2, si0, di0, smi0)

            pltpu.make_async_copy(tbl_hbm.at[si1], r1, sg1).wait()
            sc1 = pltpu.async_copy(r1, acc_sh.at[di1], ss1, add=True)

            @pl.when(j < last)
            def _():
                idx_wait(i + 2, si0, di0, smi0)
                pltpu.async_copy(tbl_hbm.at[si0], r0, sg0)

            sc1.wait()

            @pl.when(j < last)
            def _():
                idx_load(i + 3, si1, di1, smi1)

            return carry

        lax.fori_loop(0, _NPAIRS, step, 0)
        i = _NCH - 1
        idx_load(i, si0, di0, smi0)
        idx_wait(i, si0, di0, smi0)
        pltpu.sync_copy(tbl_hbm.at[si0], r0)
        pltpu.sync_copy(r0, acc_sh.at[di0], add=True)
        plsc.subcore_barrier()
        pltpu.sync_copy(acc_sh.at[pl.ds(slab, _SLAB)], vbuf)
        pltpu.sync_copy(vbuf, out_hbm.at[c, pl.ds(slab, _SLAB)])

    return agg


def _tc1_body(x_ref, wl_ref, wr_ref, b1_ref, y80_ref, yr_ref):
    x = x_ref[...]
    y = jnp.dot(x, wl_ref[...], preferred_element_type=jnp.float32,
                precision=lax.Precision.HIGHEST)
    ones = jnp.ones((N_NODES, 16), jnp.float32)
    y80 = jnp.concatenate([y, ones], axis=1)
    pad = jnp.zeros((_NPAD - N_NODES, HIDDEN + 16), jnp.float32)
    y80_ref[...] = jnp.concatenate([y80, pad], axis=0)
    yr_ref[...] = jnp.dot(x, wr_ref[...], preferred_element_type=jnp.float32,
                precision=lax.Precision.HIGHEST) + b1_ref[...]


def _bn_relu(h, g, b):
    mu = jnp.mean(h, axis=0, keepdims=True)
    var = jnp.mean((h - mu) * (h - mu), axis=0, keepdims=True)
    return jnp.maximum((h - mu) * lax.rsqrt(var + 1e-5) * g + b, 0.0)


def _tc2_body(sums_ref, yr_ref, g1_ref, be1_ref, wl_ref, wr_ref, b2_ref,
              z80_ref, zr_ref, r64_ref):
    s = sums_ref[0, :N_NODES, :] + sums_ref[1, :N_NODES, :]
    cnt = s[:, HIDDEN:HIDDEN + 16]
    r = 1.0 / jnp.maximum(cnt, 1.0)
    r64 = jnp.concatenate([r, r, r, r], axis=1)
    h = s[:, :HIDDEN] * r64 + yr_ref[...]
    h = _bn_relu(h, g1_ref[...], be1_ref[...])
    z = jnp.dot(h, wl_ref[...], preferred_element_type=jnp.float32,
                precision=lax.Precision.HIGHEST)
    pad = jnp.zeros((_NPAD - N_NODES, HIDDEN), jnp.float32)
    z80_ref[...] = jnp.concatenate([z, pad], axis=0)
    zr_ref[...] = jnp.dot(h, wr_ref[...], preferred_element_type=jnp.float32,
                precision=lax.Precision.HIGHEST) + b2_ref[...]
    r64_ref[...] = r64


def _tc3_body(sums_ref, zr_ref, r64_ref, g2_ref, be2_ref, batch_ref,
              wc1_ref, bc1_ref, gc_ref, bec_ref, wc2_ref, bc2_ref, out_ref):
    s = sums_ref[0, :N_NODES, :] + sums_ref[1, :N_NODES, :]
    h = s * r64_ref[...] + zr_ref[...]
    h = _bn_relu(h, g2_ref[...], be2_ref[...])
    gids = lax.broadcasted_iota(jnp.int32, (NUM_GRAPHS, N_NODES), 0)
    oh = (gids == batch_ref[...]).astype(jnp.float32)
    pooled = jnp.dot(oh, h, preferred_element_type=jnp.float32,
                precision=lax.Precision.HIGHEST)
    cg = jnp.sum(oh, axis=1, keepdims=True)
    pooled = pooled / jnp.maximum(cg, 1.0)
    hc = jnp.dot(pooled, wc1_ref[...], preferred_element_type=jnp.float32,
                precision=lax.Precision.HIGHEST) + bc1_ref[...]
    mu = jnp.mean(hc, axis=0, keepdims=True)
    var = jnp.mean((hc - mu) * (hc - mu), axis=0, keepdims=True)
    hc = jnp.maximum((hc - mu) * lax.rsqrt(var + 1e-5) * gc_ref[...] + bec_ref[...], 0.0)
    logits = jnp.dot(hc, wc2_ref[...], preferred_element_type=jnp.float32,
                precision=lax.Precision.HIGHEST) + bc2_ref[...]
    m = jnp.max(logits, axis=1, keepdims=True)
    lse = m + jnp.log(jnp.sum(jnp.exp(logits - m), axis=1, keepdims=True))
    out_ref[...] = logits - lse


def kernel(x, edge_index, batch, W1_l, W1_r, b1, W2_l, W2_r, b2,
           g1, be1, g2, be2, Wc1, bc1, gc, bec, Wc2, bc2):
    f32 = jnp.float32
    src = edge_index[0].astype(jnp.int32)
    dst = edge_index[1].astype(jnp.int32)
    batch2d = batch.astype(jnp.int32).reshape(1, N_NODES)
    zrow80 = jnp.zeros((_SLAB, HIDDEN + 16), f32)
    zrow64 = jnp.zeros((_SLAB, HIDDEN), f32)

    y80, y1r = pl.pallas_call(
        _tc1_body,
        out_shape=[jax.ShapeDtypeStruct((_NPAD, HIDDEN + 16), f32),
                   jax.ShapeDtypeStruct((N_NODES, HIDDEN), f32)],
    )(x, W1_l, W1_r, b1.reshape(1, HIDDEN))

    sums1 = _make_sc_agg(HIDDEN + 16)(y80, src, dst, zrow80)

    z64, z2r, r64 = pl.pallas_call(
        _tc2_body,
        out_shape=[jax.ShapeDtypeStruct((_NPAD, HIDDEN), f32),
                   jax.ShapeDtypeStruct((N_NODES, HIDDEN), f32),
                   jax.ShapeDtypeStruct((N_NODES, HIDDEN), f32)],
    )(sums1, y1r, g1.reshape(1, HIDDEN), be1.reshape(1, HIDDEN),
      W2_l, W2_r, b2.reshape(1, HIDDEN))

    sums2 = _make_sc_agg(HIDDEN)(z64, src, dst, zrow64)

    out = pl.pallas_call(
        _tc3_body,
        out_shape=jax.ShapeDtypeStruct((NUM_GRAPHS, N_CLASSES), f32),
    )(sums2, z2r, r64, g2.reshape(1, HIDDEN), be2.reshape(1, HIDDEN),
      batch2d, Wc1, bc1.reshape(1, HIDDEN), gc.reshape(1, HIDDEN),
      bec.reshape(1, HIDDEN), Wc2, bc2.reshape(1, N_CLASSES))
    return out


# speedup vs baseline: 10.6253x; 1.0536x over previous
import functools

import jax
import jax.numpy as jnp
from jax import lax
from jax.experimental import pallas as pl
from jax.experimental.pallas import tpu as pltpu
from jax.experimental.pallas import tpu_sc as plsc

N_NODES = 10000
N_EDGES = 320000
D_FEAT = 128
HIDDEN = 64
N_CLASSES = 10
NUM_GRAPHS = 64

_NC = 2
_NS = 16
_NW = _NC * _NS
_EPT = N_EDGES // _NW
_CH = 80
_NCH = _EPT // _CH
_NPAIRS = _NCH // 2
_NPAD = 10240
_SLAB = _NPAD // _NS


@functools.lru_cache(maxsize=None)
def _make_sc_agg(width, dtype=jnp.float32):
    mesh = plsc.VectorSubcoreMesh(core_axis_name="c", subcore_axis_name="s")

    @functools.partial(
        pl.kernel,
        mesh=mesh,
        compiler_params=pltpu.CompilerParams(use_tc_tiling_on_sc=False),
        out_type=jax.ShapeDtypeStruct((_NC, _NPAD, width), dtype),
        scratch_types=[
            pltpu.VMEM_SHARED((_NPAD, width), dtype),
            pltpu.VMEM((_SLAB, width), dtype),
            pltpu.VMEM((_CH,), jnp.int32),
            pltpu.VMEM((_CH,), jnp.int32),
            pltpu.VMEM((_CH,), jnp.int32),
            pltpu.VMEM((_CH,), jnp.int32),
            pltpu.VMEM((_CH, width), dtype),
            pltpu.VMEM((_CH, width), dtype),
            pltpu.SemaphoreType.DMA,
            pltpu.SemaphoreType.DMA,
            pltpu.SemaphoreType.DMA,
            pltpu.SemaphoreType.DMA,
            pltpu.SemaphoreType.DMA,
            pltpu.SemaphoreType.DMA,
        ],
    )
    def agg(tbl_hbm, src_hbm, dst_hbm, zrow_hbm, out_hbm,
            acc_sh, vbuf, si0, di0, si1, di1, r0, r1,
            smi0, smi1, sg0, sg1, ss0, ss1):
        c = lax.axis_index("c")
        s = lax.axis_index("s")
        slab = s * _SLAB
        ebase = (c * _NS + s) * _EPT
        pltpu.sync_copy(zrow_hbm, vbuf)
        pltpu.sync_copy(vbuf, acc_sh.at[pl.ds(slab, _SLAB)])
        plsc.subcore_barrier()

        def idx_load(i, sref, dref, sem):
            off = ebase + i * _CH
            pltpu.async_copy(src_hbm.at[pl.ds(off, _CH)], sref, sem)
            pltpu.async_copy(dst_hbm.at[pl.ds(off, _CH)], dref, sem)

        def idx_wait(i, sref, dref, sem):
            off = ebase + i * _CH
            pltpu.make_async_copy(src_hbm.at[pl.ds(off, _CH)], sref, sem).wait()
            pltpu.make_async_copy(dst_hbm.at[pl.ds(off, _CH)], dref, sem).wait()

        idx_load(0, si0, di0, smi0)
        idx_load(1, si1, di1, smi1)
        idx_wait(0, si0, di0, smi0)
        pltpu.async_copy(tbl_hbm.at[si0], r0, sg0)
        last = _NPAIRS - 1

        def step(j, carry):
            i = 2 * j
            pltpu.make_async_copy(tbl_hbm.at[si0], r0, sg0).wait()
            sc0 = pltpu.async_copy(r0, acc_sh.at[di0], ss0, add=True)
            idx_wait(i + 1, si1, di1, smi1)
            pltpu.async_copy(tbl_hbm.at[si1], r1, sg1)
            sc0.wait()

            @pl.when(j < last)
            def _():
                idx_load(i + 2, si0, di0, smi0)

            pltpu.make_async_copy(tbl_hbm.at[si1], r1, sg1).wait()
            sc1 = pltpu.async_copy(r1, acc_sh.at[di1], ss1, add=True)

            @pl.when(j < last)
            def _():
                idx_wait(i + 2, si0, di0, smi0)
                pltpu.async_copy(tbl_hbm.at[si0], r0, sg0)

            sc1.wait()

            @pl.when(j < last)
            def _():
                idx_load(i + 3, si1, di1, smi1)

            return carry

        lax.fori_loop(0, _NPAIRS, step, 0)
        i = _NCH - 1
        idx_load(i, si0, di0, smi0)
        idx_wait(i, si0, di0, smi0)
        pltpu.sync_copy(tbl_hbm.at[si0], r0)
        pltpu.sync_copy(r0, acc_sh.at[di0], add=True)
        plsc.subcore_barrier()
        pltpu.sync_copy(acc_sh.at[pl.ds(slab, _SLAB)], vbuf)
        pltpu.sync_copy(vbuf, out_hbm.at[c, pl.ds(slab, _SLAB)])

    return agg


def _tc1_body(x_ref, wl_ref, wr_ref, b1_ref, y80_ref, yr_ref):
    x = x_ref[...]
    y = jnp.dot(x, wl_ref[...], preferred_element_type=jnp.float32,
                precision=lax.Precision.HIGHEST)
    ones = jnp.ones((N_NODES, 16), jnp.float32)
    y80 = jnp.concatenate([y, ones], axis=1)
    pad = jnp.zeros((_NPAD - N_NODES, HIDDEN + 16), jnp.float32)
    y80_ref[...] = jnp.concatenate([y80, pad], axis=0)
    yr_ref[...] = jnp.dot(x, wr_ref[...], preferred_element_type=jnp.float32,
                precision=lax.Precision.HIGHEST) + b1_ref[...]


def _bn_relu(h, g, b):
    mu = jnp.mean(h, axis=0, keepdims=True)
    var = jnp.mean((h - mu) * (h - mu), axis=0, keepdims=True)
    return jnp.maximum((h - mu) * lax.rsqrt(var + 1e-5) * g + b, 0.0)


def _tc2_body(sums_ref, yr_ref, g1_ref, be1_ref, wl_ref, wr_ref, b2_ref,
              z80_ref, zr_ref, r64_ref):
    s = sums_ref[0, :N_NODES, :] + sums_ref[1, :N_NODES, :]
    cnt = s[:, HIDDEN:HIDDEN + 16]
    r = 1.0 / jnp.maximum(cnt, 1.0)
    r64 = jnp.concatenate([r, r, r, r], axis=1)
    h = s[:, :HIDDEN] * r64 + yr_ref[...]
    h = _bn_relu(h, g1_ref[...], be1_ref[...])
    z = jnp.dot(h, wl_ref[...], preferred_element_type=jnp.float32,
                precision=lax.Precision.HIGHEST)
    pad = jnp.zeros((_NPAD - N_NODES, HIDDEN), jnp.float32)
    z80_ref[...] = jnp.concatenate([z, pad], axis=0).astype(jnp.bfloat16)
    zr_ref[...] = jnp.dot(h, wr_ref[...], preferred_element_type=jnp.float32,
                precision=lax.Precision.HIGHEST) + b2_ref[...]
    r64_ref[...] = r64


def _tc3_body(sums_ref, zr_ref, r64_ref, g2_ref, be2_ref, batch_ref,
              wc1_ref, bc1_ref, gc_ref, bec_ref, wc2_ref, bc2_ref, out_ref):
    s = (sums_ref[0, :N_NODES, :].astype(jnp.float32)
         + sums_ref[1, :N_NODES, :].astype(jnp.float32))
    h = s * r64_ref[...] + zr_ref[...]
    h = _bn_relu(h, g2_ref[...], be2_ref[...])
    gids = lax.broadcasted_iota(jnp.int32, (NUM_GRAPHS, N_NODES), 0)
    oh = (gids == batch_ref[...]).astype(jnp.float32)
    pooled = jnp.dot(oh, h, preferred_element_type=jnp.float32,
                precision=lax.Precision.HIGHEST)
    cg = jnp.sum(oh, axis=1, keepdims=True)
    pooled = pooled / jnp.maximum(cg, 1.0)
    hc = jnp.dot(pooled, wc1_ref[...], preferred_element_type=jnp.float32,
                precision=lax.Precision.HIGHEST) + bc1_ref[...]
    mu = jnp.mean(hc, axis=0, keepdims=True)
    var = jnp.mean((hc - mu) * (hc - mu), axis=0, keepdims=True)
    hc = jnp.maximum((hc - mu) * lax.rsqrt(var + 1e-5) * gc_ref[...] + bec_ref[...], 0.0)
    logits = jnp.dot(hc, wc2_ref[...], preferred_element_type=jnp.float32,
                precision=lax.Precision.HIGHEST) + bc2_ref[...]
    m = jnp.max(logits, axis=1, keepdims=True)
    lse = m + jnp.log(jnp.sum(jnp.exp(logits - m), axis=1, keepdims=True))
    out_ref[...] = logits - lse


def kernel(x, edge_index, batch, W1_l, W1_r, b1, W2_l, W2_r, b2,
           g1, be1, g2, be2, Wc1, bc1, gc, bec, Wc2, bc2):
    f32 = jnp.float32
    src = edge_index[0].astype(jnp.int32)
    dst = edge_index[1].astype(jnp.int32)
    batch2d = batch.astype(jnp.int32).reshape(1, N_NODES)
    zrow80 = jnp.zeros((_SLAB, HIDDEN + 16), f32)
    zrow64 = jnp.zeros((_SLAB, HIDDEN), jnp.bfloat16)

    y80, y1r = pl.pallas_call(
        _tc1_body,
        out_shape=[jax.ShapeDtypeStruct((_NPAD, HIDDEN + 16), f32),
                   jax.ShapeDtypeStruct((N_NODES, HIDDEN), f32)],
    )(x, W1_l, W1_r, b1.reshape(1, HIDDEN))

    sums1 = _make_sc_agg(HIDDEN + 16)(y80, src, dst, zrow80)

    z64, z2r, r64 = pl.pallas_call(
        _tc2_body,
        out_shape=[jax.ShapeDtypeStruct((_NPAD, HIDDEN), jnp.bfloat16),
                   jax.ShapeDtypeStruct((N_NODES, HIDDEN), f32),
                   jax.ShapeDtypeStruct((N_NODES, HIDDEN), f32)],
    )(sums1, y1r, g1.reshape(1, HIDDEN), be1.reshape(1, HIDDEN),
      W2_l, W2_r, b2.reshape(1, HIDDEN))

    sums2 = _make_sc_agg(HIDDEN, jnp.bfloat16)(z64, src, dst, zrow64)

    out = pl.pallas_call(
        _tc3_body,
        out_shape=jax.ShapeDtypeStruct((NUM_GRAPHS, N_CLASSES), f32),
    )(sums2, z2r, r64, g2.reshape(1, HIDDEN), be2.reshape(1, HIDDEN),
      batch2d, Wc1, bc1.reshape(1, HIDDEN), gc.reshape(1, HIDDEN),
      bec.reshape(1, HIDDEN), Wc2, bc2.reshape(1, N_CLASSES))
    return out


# speedup vs baseline: 10.8462x; 1.0208x over previous
import functools

import jax
import jax.numpy as jnp
from jax import lax
from jax.experimental import pallas as pl
from jax.experimental.pallas import tpu as pltpu
from jax.experimental.pallas import tpu_sc as plsc

N_NODES = 10000
N_EDGES = 320000
D_FEAT = 128
HIDDEN = 64
N_CLASSES = 10
NUM_GRAPHS = 64

_NC = 2
_NS = 16
_NW = _NC * _NS
_EPT = N_EDGES // _NW
_CH = 80
_NCH = _EPT // _CH
_NPAIRS = _NCH // 2
_NPAD = 10240
_SLAB = _NPAD // _NS


@functools.lru_cache(maxsize=None)
def _make_sc_agg(width, dtype=jnp.float32):
    mesh = plsc.VectorSubcoreMesh(core_axis_name="c", subcore_axis_name="s")

    @functools.partial(
        pl.kernel,
        mesh=mesh,
        compiler_params=pltpu.CompilerParams(use_tc_tiling_on_sc=False),
        out_type=jax.ShapeDtypeStruct((_NC, _NPAD, width), dtype),
        scratch_types=[
            pltpu.VMEM_SHARED((_NPAD, width), dtype),
            pltpu.VMEM((_SLAB, width), dtype),
            pltpu.VMEM((_CH,), jnp.int32),
            pltpu.VMEM((_CH,), jnp.int32),
            pltpu.VMEM((_CH,), jnp.int32),
            pltpu.VMEM((_CH,), jnp.int32),
            pltpu.VMEM((_CH, width), dtype),
            pltpu.VMEM((_CH, width), dtype),
            pltpu.SemaphoreType.DMA,
            pltpu.SemaphoreType.DMA,
            pltpu.SemaphoreType.DMA,
            pltpu.SemaphoreType.DMA,
            pltpu.SemaphoreType.DMA,
            pltpu.SemaphoreType.DMA,
        ],
    )
    def agg(tbl_hbm, src_hbm, dst_hbm, zrow_hbm, out_hbm,
            acc_sh, vbuf, si0, di0, si1, di1, r0, r1,
            smi0, smi1, sg0, sg1, ss0, ss1):
        c = lax.axis_index("c")
        s = lax.axis_index("s")
        slab = s * _SLAB
        ebase = (c * _NS + s) * _EPT
        pltpu.sync_copy(zrow_hbm, vbuf)
        pltpu.sync_copy(vbuf, acc_sh.at[pl.ds(slab, _SLAB)])
        plsc.subcore_barrier()

        def idx_load(i, sref, dref, sem):
            off = ebase + i * _CH
            pltpu.async_copy(src_hbm.at[pl.ds(off, _CH)], sref, sem)
            pltpu.async_copy(dst_hbm.at[pl.ds(off, _CH)], dref, sem)

        def idx_wait(i, sref, dref, sem):
            off = ebase + i * _CH
            pltpu.make_async_copy(src_hbm.at[pl.ds(off, _CH)], sref, sem).wait()
            pltpu.make_async_copy(dst_hbm.at[pl.ds(off, _CH)], dref, sem).wait()

        idx_load(0, si0, di0, smi0)
        idx_load(1, si1, di1, smi1)
        idx_wait(0, si0, di0, smi0)
        pltpu.async_copy(tbl_hbm.at[si0], r0, sg0)
        last = _NPAIRS - 1

        def step(j, carry):
            i = 2 * j
            pltpu.make_async_copy(tbl_hbm.at[si0], r0, sg0).wait()
            sc0 = pltpu.async_copy(r0, acc_sh.at[di0], ss0, add=True)
            idx_wait(i + 1, si1, di1, smi1)
            pltpu.async_copy(tbl_hbm.at[si1], r1, sg1)
            sc0.wait()

            @pl.when(j < last)
            def _():
                idx_load(i + 2, si0, di0, smi0)

            pltpu.make_async_copy(tbl_hbm.at[si1], r1, sg1).wait()
            sc1 = pltpu.async_copy(r1, acc_sh.at[di1], ss1, add=True)

            @pl.when(j < last)
            def _():
                idx_wait(i + 2, si0, di0, smi0)
                pltpu.async_copy(tbl_hbm.at[si0], r0, sg0)

            sc1.wait()

            @pl.when(j < last)
            def _():
                idx_load(i + 3, si1, di1, smi1)

            return carry

        lax.fori_loop(0, _NPAIRS, step, 0)
        i = _NCH - 1
        idx_load(i, si0, di0, smi0)
        idx_wait(i, si0, di0, smi0)
        pltpu.sync_copy(tbl_hbm.at[si0], r0)
        pltpu.sync_copy(r0, acc_sh.at[di0], add=True)
        plsc.subcore_barrier()
        pltpu.sync_copy(acc_sh.at[pl.ds(slab, _SLAB)], vbuf)
        pltpu.sync_copy(vbuf, out_hbm.at[c, pl.ds(slab, _SLAB)])

    return agg


def _tc1_body(x_ref, wl_ref, wr_ref, b1_ref, y96_ref, yr_ref):
    x = x_ref[...]
    y = jnp.dot(x, wl_ref[...], preferred_element_type=jnp.float32,
                precision=lax.Precision.HIGHEST)
    ones = jnp.ones((N_NODES, 32), jnp.float32)
    y96 = jnp.concatenate([y, ones], axis=1)
    pad = jnp.zeros((_NPAD - N_NODES, HIDDEN + 32), jnp.float32)
    y96_ref[...] = jnp.concatenate([y96, pad], axis=0).astype(jnp.bfloat16)
    yr_ref[...] = jnp.dot(x, wr_ref[...], preferred_element_type=jnp.float32,
                precision=lax.Precision.HIGHEST) + b1_ref[...]


def _bn_relu(h, g, b):
    mu = jnp.mean(h, axis=0, keepdims=True)
    var = jnp.mean((h - mu) * (h - mu), axis=0, keepdims=True)
    return jnp.maximum((h - mu) * lax.rsqrt(var + 1e-5) * g + b, 0.0)


def _tc2_body(sums_ref, yr_ref, g1_ref, be1_ref, wl_ref, wr_ref, b2_ref,
              z80_ref, zr_ref, r64_ref):
    s = (sums_ref[0, :N_NODES, :].astype(jnp.float32)
         + sums_ref[1, :N_NODES, :].astype(jnp.float32))
    cnt = s[:, HIDDEN:HIDDEN + 16]
    r = 1.0 / jnp.maximum(cnt, 1.0)
    r64 = jnp.concatenate([r, r, r, r], axis=1)
    h = s[:, :HIDDEN] * r64 + yr_ref[...]
    h = _bn_relu(h, g1_ref[...], be1_ref[...])
    z = jnp.dot(h, wl_ref[...], preferred_element_type=jnp.float32,
                precision=lax.Precision.HIGHEST)
    pad = jnp.zeros((_NPAD - N_NODES, HIDDEN), jnp.float32)
    z80_ref[...] = jnp.concatenate([z, pad], axis=0).astype(jnp.bfloat16)
    zr_ref[...] = jnp.dot(h, wr_ref[...], preferred_element_type=jnp.float32,
                precision=lax.Precision.HIGHEST) + b2_ref[...]
    r64_ref[...] = r64


def _tc3_body(sums_ref, zr_ref, r64_ref, g2_ref, be2_ref, batch_ref,
              wc1_ref, bc1_ref, gc_ref, bec_ref, wc2_ref, bc2_ref, out_ref):
    s = (sums_ref[0, :N_NODES, :].astype(jnp.float32)
         + sums_ref[1, :N_NODES, :].astype(jnp.float32))
    h = s * r64_ref[...] + zr_ref[...]
    h = _bn_relu(h, g2_ref[...], be2_ref[...])
    gids = lax.broadcasted_iota(jnp.int32, (NUM_GRAPHS, N_NODES), 0)
    oh = (gids == batch_ref[...]).astype(jnp.float32)
    pooled = jnp.dot(oh, h, preferred_element_type=jnp.float32,
                precision=lax.Precision.HIGHEST)
    cg = jnp.sum(oh, axis=1, keepdims=True)
    pooled = pooled / jnp.maximum(cg, 1.0)
    hc = jnp.dot(pooled, wc1_ref[...], preferred_element_type=jnp.float32,
                precision=lax.Precision.HIGHEST) + bc1_ref[...]
    mu = jnp.mean(hc, axis=0, keepdims=True)
    var = jnp.mean((hc - mu) * (hc - mu), axis=0, keepdims=True)
    hc = jnp.maximum((hc - mu) * lax.rsqrt(var + 1e-5) * gc_ref[...] + bec_ref[...], 0.0)
    logits = jnp.dot(hc, wc2_ref[...], preferred_element_type=jnp.float32,
                precision=lax.Precision.HIGHEST) + bc2_ref[...]
    m = jnp.max(logits, axis=1, keepdims=True)
    lse = m + jnp.log(jnp.sum(jnp.exp(logits - m), axis=1, keepdims=True))
    out_ref[...] = logits - lse


def kernel(x, edge_index, batch, W1_l, W1_r, b1, W2_l, W2_r, b2,
           g1, be1, g2, be2, Wc1, bc1, gc, bec, Wc2, bc2):
    f32 = jnp.float32
    src = edge_index[0].astype(jnp.int32)
    dst = edge_index[1].astype(jnp.int32)
    batch2d = batch.astype(jnp.int32).reshape(1, N_NODES)
    zrow96 = jnp.zeros((_SLAB, HIDDEN + 32), jnp.bfloat16)
    zrow64 = jnp.zeros((_SLAB, HIDDEN), jnp.bfloat16)

    y96, y1r = pl.pallas_call(
        _tc1_body,
        out_shape=[jax.ShapeDtypeStruct((_NPAD, HIDDEN + 32), jnp.bfloat16),
                   jax.ShapeDtypeStruct((N_NODES, HIDDEN), f32)],
    )(x, W1_l, W1_r, b1.reshape(1, HIDDEN))

    sums1 = _make_sc_agg(HIDDEN + 32, jnp.bfloat16)(y96, src, dst, zrow96)

    z64, z2r, r64 = pl.pallas_call(
        _tc2_body,
        out_shape=[jax.ShapeDtypeStruct((_NPAD, HIDDEN), jnp.bfloat16),
                   jax.ShapeDtypeStruct((N_NODES, HIDDEN), f32),
                   jax.ShapeDtypeStruct((N_NODES, HIDDEN), f32)],
    )(sums1, y1r, g1.reshape(1, HIDDEN), be1.reshape(1, HIDDEN),
      W2_l, W2_r, b2.reshape(1, HIDDEN))

    sums2 = _make_sc_agg(HIDDEN, jnp.bfloat16)(z64, src, dst, zrow64)

    out = pl.pallas_call(
        _tc3_body,
        out_shape=jax.ShapeDtypeStruct((NUM_GRAPHS, N_CLASSES), f32),
    )(sums2, z2r, r64, g2.reshape(1, HIDDEN), be2.reshape(1, HIDDEN),
      batch2d, Wc1, bc1.reshape(1, HIDDEN), gc.reshape(1, HIDDEN),
      bec.reshape(1, HIDDEN), Wc2, bc2.reshape(1, N_CLASSES))
    return out


# speedup vs baseline: 12.7125x; 1.1721x over previous
import functools

import jax
import jax.numpy as jnp
from jax import lax
from jax.experimental import pallas as pl
from jax.experimental.pallas import tpu as pltpu
from jax.experimental.pallas import tpu_sc as plsc

N_NODES = 10000
N_EDGES = 320000
D_FEAT = 128
HIDDEN = 64
N_CLASSES = 10
NUM_GRAPHS = 64

_NC = 2
_NS = 16
_NW = _NC * _NS
_EPT = N_EDGES // _NW
_CH = 128
_NCH = 78
_NPAIRS = _NCH // 2
_TAIL = _EPT - _NCH * _CH
_EROWS = N_EDGES // _CH
_NPAD = 10240
_SLAB = _NPAD // _NS


@functools.lru_cache(maxsize=None)
def _make_sc_agg(width, dtype=jnp.float32):
    mesh = plsc.VectorSubcoreMesh(core_axis_name="c", subcore_axis_name="s")

    @functools.partial(
        pl.kernel,
        mesh=mesh,
        compiler_params=pltpu.CompilerParams(use_tc_tiling_on_sc=False),
        out_type=jax.ShapeDtypeStruct((_NC, _NPAD, width), dtype),
        scratch_types=[
            pltpu.VMEM_SHARED((_NPAD, width), dtype),
            pltpu.VMEM((_SLAB, width), dtype),
            pltpu.VMEM((_NCH, _CH), jnp.int32),
            pltpu.VMEM((_NCH, _CH), jnp.int32),
            pltpu.VMEM((_TAIL,), jnp.int32),
            pltpu.VMEM((_TAIL,), jnp.int32),
            pltpu.VMEM((_CH, width), dtype),
            pltpu.VMEM((_CH, width), dtype),
            pltpu.VMEM((_TAIL, width), dtype),
            pltpu.SemaphoreType.DMA,
            pltpu.SemaphoreType.DMA,
            pltpu.SemaphoreType.DMA,
            pltpu.SemaphoreType.DMA,
        ],
    )
    def agg(tbl_hbm, src_hbm, dst_hbm, zrow_hbm, out_hbm,
            acc_sh, vbuf, sblk, dblk, st, dt, r0, r1, rt,
            sg0, sg1, ss0, ss1):
        c = lax.axis_index("c")
        s = lax.axis_index("s")
        slab = s * _SLAB
        wid = c * _NS + s
        pltpu.sync_copy(zrow_hbm, vbuf)
        pltpu.sync_copy(vbuf, acc_sh.at[pl.ds(slab, _SLAB)])
        pltpu.sync_copy(src_hbm.at[pl.ds(wid * _NCH, _NCH)], sblk)
        pltpu.sync_copy(dst_hbm.at[pl.ds(wid * _NCH, _NCH)], dblk)
        tr = _NW * _NCH + wid * _TAIL // _CH
        tc0 = (wid * _TAIL) % _CH
        pltpu.sync_copy(src_hbm.at[tr, pl.ds(tc0, _TAIL)], st)
        pltpu.sync_copy(dst_hbm.at[tr, pl.ds(tc0, _TAIL)], dt)
        plsc.subcore_barrier()

        pltpu.async_copy(tbl_hbm.at[sblk.at[0]], r0, sg0)

        def step(j, carry):
            i = 2 * j
            pltpu.make_async_copy(tbl_hbm.at[sblk.at[i]], r0, sg0).wait()
            sc0 = pltpu.async_copy(r0, acc_sh.at[dblk.at[i]], ss0, add=True)
            pltpu.async_copy(tbl_hbm.at[sblk.at[i + 1]], r1, sg1)
            sc0.wait()
            pltpu.make_async_copy(tbl_hbm.at[sblk.at[i + 1]], r1, sg1).wait()
            sc1 = pltpu.async_copy(r1, acc_sh.at[dblk.at[i + 1]], ss1, add=True)

            @pl.when(j < _NPAIRS - 1)
            def _():
                pltpu.async_copy(tbl_hbm.at[sblk.at[i + 2]], r0, sg0)

            sc1.wait()
            return carry

        lax.fori_loop(0, _NPAIRS, step, 0)
        pltpu.sync_copy(tbl_hbm.at[st], rt)
        pltpu.sync_copy(rt, acc_sh.at[dt], add=True)
        plsc.subcore_barrier()
        pltpu.sync_copy(acc_sh.at[pl.ds(slab, _SLAB)], vbuf)
        pltpu.sync_copy(vbuf, out_hbm.at[c, pl.ds(slab, _SLAB)])

    return agg


def _tc1_body(x_ref, wl_ref, wr_ref, b1_ref, y96_ref, yr_ref):
    x = x_ref[...]
    y = jnp.dot(x, wl_ref[...], preferred_element_type=jnp.float32,
                precision=lax.Precision.HIGHEST)
    ones = jnp.ones((N_NODES, 32), jnp.float32)
    y96 = jnp.concatenate([y, ones], axis=1)
    pad = jnp.zeros((_NPAD - N_NODES, HIDDEN + 32), jnp.float32)
    y96_ref[...] = jnp.concatenate([y96, pad], axis=0).astype(jnp.bfloat16)
    yr_ref[...] = jnp.dot(x, wr_ref[...], preferred_element_type=jnp.float32,
                precision=lax.Precision.HIGHEST) + b1_ref[...]


def _bn_relu(h, g, b):
    mu = jnp.mean(h, axis=0, keepdims=True)
    var = jnp.mean((h - mu) * (h - mu), axis=0, keepdims=True)
    return jnp.maximum((h - mu) * lax.rsqrt(var + 1e-5) * g + b, 0.0)


def _tc2_body(sums_ref, yr_ref, g1_ref, be1_ref, wl_ref, wr_ref, b2_ref,
              z80_ref, zr_ref, r64_ref):
    s = (sums_ref[0, :N_NODES, :].astype(jnp.float32)
         + sums_ref[1, :N_NODES, :].astype(jnp.float32))
    cnt = s[:, HIDDEN:HIDDEN + 16]
    r = 1.0 / jnp.maximum(cnt, 1.0)
    r64 = jnp.concatenate([r, r, r, r], axis=1)
    h = s[:, :HIDDEN] * r64 + yr_ref[...]
    h = _bn_relu(h, g1_ref[...], be1_ref[...])
    z = jnp.dot(h, wl_ref[...], preferred_element_type=jnp.float32,
                precision=lax.Precision.HIGHEST)
    pad = jnp.zeros((_NPAD - N_NODES, HIDDEN), jnp.float32)
    z80_ref[...] = jnp.concatenate([z, pad], axis=0).astype(jnp.bfloat16)
    zr_ref[...] = jnp.dot(h, wr_ref[...], preferred_element_type=jnp.float32,
                precision=lax.Precision.HIGHEST) + b2_ref[...]
    r64_ref[...] = r64


def _tc3_body(sums_ref, zr_ref, r64_ref, g2_ref, be2_ref, batch_ref,
              wc1_ref, bc1_ref, gc_ref, bec_ref, wc2_ref, bc2_ref, out_ref):
    s = (sums_ref[0, :N_NODES, :].astype(jnp.float32)
         + sums_ref[1, :N_NODES, :].astype(jnp.float32))
    h = s * r64_ref[...] + zr_ref[...]
    h = _bn_relu(h, g2_ref[...], be2_ref[...])
    gids = lax.broadcasted_iota(jnp.int32, (NUM_GRAPHS, N_NODES), 0)
    oh = (gids == batch_ref[...]).astype(jnp.float32)
    pooled = jnp.dot(oh, h, preferred_element_type=jnp.float32,
                precision=lax.Precision.HIGHEST)
    cg = jnp.sum(oh, axis=1, keepdims=True)
    pooled = pooled / jnp.maximum(cg, 1.0)
    hc = jnp.dot(pooled, wc1_ref[...], preferred_element_type=jnp.float32,
                precision=lax.Precision.HIGHEST) + bc1_ref[...]
    mu = jnp.mean(hc, axis=0, keepdims=True)
    var = jnp.mean((hc - mu) * (hc - mu), axis=0, keepdims=True)
    hc = jnp.maximum((hc - mu) * lax.rsqrt(var + 1e-5) * gc_ref[...] + bec_ref[...], 0.0)
    logits = jnp.dot(hc, wc2_ref[...], preferred_element_type=jnp.float32,
                precision=lax.Precision.HIGHEST) + bc2_ref[...]
    m = jnp.max(logits, axis=1, keepdims=True)
    lse = m + jnp.log(jnp.sum(jnp.exp(logits - m), axis=1, keepdims=True))
    out_ref[...] = logits - lse


def kernel(x, edge_index, batch, W1_l, W1_r, b1, W2_l, W2_r, b2,
           g1, be1, g2, be2, Wc1, bc1, gc, bec, Wc2, bc2):
    f32 = jnp.float32
    src = edge_index[0].astype(jnp.int32).reshape(_EROWS, _CH)
    dst = edge_index[1].astype(jnp.int32).reshape(_EROWS, _CH)
    batch2d = batch.astype(jnp.int32).reshape(1, N_NODES)
    zrow96 = jnp.zeros((_SLAB, HIDDEN + 32), jnp.bfloat16)
    zrow64 = jnp.zeros((_SLAB, HIDDEN), jnp.bfloat16)

    y96, y1r = pl.pallas_call(
        _tc1_body,
        out_shape=[jax.ShapeDtypeStruct((_NPAD, HIDDEN + 32), jnp.bfloat16),
                   jax.ShapeDtypeStruct((N_NODES, HIDDEN), f32)],
    )(x, W1_l, W1_r, b1.reshape(1, HIDDEN))

    sums1 = _make_sc_agg(HIDDEN + 32, jnp.bfloat16)(y96, src, dst, zrow96)

    z64, z2r, r64 = pl.pallas_call(
        _tc2_body,
        out_shape=[jax.ShapeDtypeStruct((_NPAD, HIDDEN), jnp.bfloat16),
                   jax.ShapeDtypeStruct((N_NODES, HIDDEN), f32),
                   jax.ShapeDtypeStruct((N_NODES, HIDDEN), f32)],
    )(sums1, y1r, g1.reshape(1, HIDDEN), be1.reshape(1, HIDDEN),
      W2_l, W2_r, b2.reshape(1, HIDDEN))

    sums2 = _make_sc_agg(HIDDEN, jnp.bfloat16)(z64, src, dst, zrow64)

    out = pl.pallas_call(
        _tc3_body,
        out_shape=jax.ShapeDtypeStruct((NUM_GRAPHS, N_CLASSES), f32),
    )(sums2, z2r, r64, g2.reshape(1, HIDDEN), be2.reshape(1, HIDDEN),
      batch2d, Wc1, bc1.reshape(1, HIDDEN), gc.reshape(1, HIDDEN),
      bec.reshape(1, HIDDEN), Wc2, bc2.reshape(1, N_CLASSES))
    return out
